# baseline (speedup 1.0000x reference)
import functools

import jax
import jax.numpy as jnp
from jax import lax
from jax.experimental import pallas as pl
from jax.experimental.pallas import tpu as pltpu
from jax.experimental.pallas import tpu_sc as plsc

_NC = 2
_NS = 16
_NW = _NC * _NS
_C = 80
_MB = 512


def _sc_mesh():
    return plsc.VectorSubcoreMesh(core_axis_name="c", subcore_axis_name="s")


def _sub_slice(s, rps):
    return pl.ds(pl.multiple_of(s * rps, 8), rps)


_NB = 2


def _pipelined_edge_pass(table_hbm, src_v, dst_v, row_v, gsems, acc,
                         w_chunks, c_w):
    def src_sl(jj):
        return src_v.at[pl.ds(pl.multiple_of(jj * c_w, 8), c_w)]

    def start(jj, sub):
        pltpu.async_copy(table_hbm.at[src_sl(jj)], row_v.at[sub],
                         gsems[sub])

    def wait(jj, sub):
        pltpu.make_async_copy(table_hbm.at[src_sl(jj)], row_v.at[sub],
                              gsems[sub]).wait()

    for sub in range(_NB):
        start(sub, sub)

    main = (w_chunks // _NB) * _NB

    @pl.loop(0, main, step=_NB)
    def _(j):
        for sub in range(_NB):
            jj = j + sub
            wait(jj, sub)
            pltpu.sync_copy(row_v.at[sub], acc.at[dst_v.at[jj]], add=True)

            @pl.when(jj + _NB < w_chunks)
            def _():
                start(jj + _NB, sub)

    for sub in range(w_chunks - main):
        jj = main + sub
        wait(jj, sub)
        pltpu.sync_copy(row_v.at[sub], acc.at[dst_v.at[jj]], add=True)


def _sc_agg_first(x, src_flat, dst3d, zeros_x):
    n, f = x.shape
    _, w_chunks, c_w = dst3d.shape
    ew = w_chunks * c_w
    rps = n // _NS

    @functools.partial(
        pl.kernel,
        out_type=jax.ShapeDtypeStruct((_NC, n, f), jnp.float32),
        mesh=_sc_mesh(),
        scratch_types=[
            pltpu.VMEM_SHARED((n, f), jnp.float32),
            pltpu.VMEM((ew,), jnp.int32),
            pltpu.VMEM((w_chunks, c_w), jnp.int32),
            pltpu.VMEM((_NB, c_w, f), jnp.float32),
        ] + [pltpu.SemaphoreType.DMA] * _NB,
    )
    def k(x_hbm, src_hbm, dst_hbm, zx_hbm,
          aggp_hbm, acc_x, src_v, dst_v, row_v, *gsems):
        c = lax.axis_index("c")
        s = lax.axis_index("s")
        wid = c * _NS + s
        sl = _sub_slice(s, rps)
        pltpu.sync_copy(zx_hbm, acc_x.at[sl])
        pltpu.sync_copy(
            src_hbm.at[pl.ds(pl.multiple_of(wid * ew, 8), ew)], src_v)
        pltpu.sync_copy(dst_hbm.at[wid], dst_v)
        plsc.subcore_barrier()

        _pipelined_edge_pass(x_hbm, src_v, dst_v, row_v, gsems, acc_x,
                             w_chunks, c_w)

        plsc.subcore_barrier()
        pltpu.sync_copy(acc_x.at[sl], aggp_hbm.at[c].at[sl])

    return k(x, src_flat, dst3d, zeros_x)


def _sc_counts(dst3d, zeros_c, ones_c, n):
    _, w_chunks, c_w = dst3d.shape
    w_cnt = ones_c.shape[1]
    rps = n // _NS

    @functools.partial(
        pl.kernel,
        out_type=jax.ShapeDtypeStruct((_NC, n, w_cnt), jnp.float32),
        mesh=_sc_mesh(),
        scratch_types=[
            pltpu.VMEM_SHARED((n, w_cnt), jnp.float32),
            pltpu.VMEM((w_chunks, c_w), jnp.int32),
            pltpu.VMEM((c_w, w_cnt), jnp.float32),
        ],
    )
    def k(dst_hbm, zc_hbm, ones_hbm, cntp_hbm, acc_c, dst_v, ones_v):
        c = lax.axis_index("c")
        s = lax.axis_index("s")
        wid = c * _NS + s
        sl = _sub_slice(s, rps)
        pltpu.sync_copy(zc_hbm, acc_c.at[sl])
        pltpu.sync_copy(ones_hbm, ones_v)
        pltpu.sync_copy(dst_hbm.at[wid], dst_v)
        plsc.subcore_barrier()

        @pl.loop(0, w_chunks)
        def _(j):
            pltpu.sync_copy(ones_v, acc_c.at[dst_v.at[j]], add=True)

        plsc.subcore_barrier()
        pltpu.sync_copy(acc_c.at[sl], cntp_hbm.at[c].at[sl])

    return k(dst3d, zeros_c, ones_c)


def _sc_agg_second(p8, src_flat, dst3d, zeros_x):
    ncj, n, f = p8.shape
    _, w_chunks, c_w = dst3d.shape
    ew = w_chunks * c_w
    rps = n // _NS

    @functools.partial(
        pl.kernel,
        out_type=jax.ShapeDtypeStruct((_NC, ncj, n, f), jnp.float32),
        mesh=_sc_mesh(),
        scratch_types=[
            pltpu.VMEM_SHARED((n, f), jnp.float32),
            pltpu.VMEM((ew,), jnp.int32),
            pltpu.VMEM((w_chunks, c_w), jnp.int32),
            pltpu.VMEM((_NB, c_w, f), jnp.float32),
        ] + [pltpu.SemaphoreType.DMA] * _NB,
    )
    def k(p8_hbm, src_hbm, dst_hbm, zx_hbm, out_hbm,
          acc, src_v, dst_v, row_v, *gsems):
        c = lax.axis_index("c")
        s = lax.axis_index("s")
        wid = c * _NS + s
        sl = _sub_slice(s, rps)
        pltpu.sync_copy(
            src_hbm.at[pl.ds(pl.multiple_of(wid * ew, 8), ew)], src_v)
        pltpu.sync_copy(dst_hbm.at[wid], dst_v)
        for cj in range(ncj):
            pltpu.sync_copy(zx_hbm, acc.at[sl])
            plsc.subcore_barrier()

            _pipelined_edge_pass(p8_hbm.at[cj], src_v, dst_v, row_v,
                                 gsems, acc, w_chunks, c_w)

            plsc.subcore_barrier()
            pltpu.sync_copy(acc.at[sl], out_hbm.at[c].at[cj].at[sl])
            plsc.subcore_barrier()

    return k(p8, src_flat, dst3d, zeros_x)


def _tc_layer1_project(x, aggp, cntp, w1l, b1, w1r, w2l, w2r):
    n, f = x.shape
    h1d = w1l.shape[1]
    h2d = w2l.shape[1]
    ncj = h2d // 128
    grid = (n // _MB,)

    def body(x_ref, aggp_ref, cntp_ref, w1l_ref, b1_ref, w1r_ref,
             w2l_ref, w2r_ref, p8_ref, q_ref):
        cnt = cntp_ref[0, :, 0:1] + cntp_ref[1, :, 0:1]
        inv = 1.0 / jnp.maximum(cnt, 1.0)
        mean1 = ((aggp_ref[0] + aggp_ref[1]) * inv).astype(jnp.bfloat16)
        h1 = jnp.maximum(
            jnp.dot(mean1, w1l_ref[...], preferred_element_type=jnp.float32)
            + jnp.dot(x_ref[...].astype(jnp.bfloat16), w1r_ref[...],
                      preferred_element_type=jnp.float32)
            + b1_ref[...], 0.0).astype(jnp.bfloat16)
        q_ref[...] = jnp.dot(h1, w2r_ref[...], preferred_element_type=jnp.float32)
        p = jnp.dot(h1, w2l_ref[...], preferred_element_type=jnp.float32)
        for cj in range(ncj):
            p8_ref[cj] = p[:, cj * 128:(cj + 1) * 128]

    return pl.pallas_call(
        body,
        grid=grid,
        in_specs=[
            pl.BlockSpec((_MB, f), lambda m: (m, 0)),
            pl.BlockSpec((_NC, _MB, f), lambda m: (0, m, 0)),
            pl.BlockSpec((_NC, _MB, 128), lambda m: (0, m, 0)),
            pl.BlockSpec((f, h1d), lambda m: (0, 0)),
            pl.BlockSpec((1, h1d), lambda m: (0, 0)),
            pl.BlockSpec((f, h1d), lambda m: (0, 0)),
            pl.BlockSpec((h1d, h2d), lambda m: (0, 0)),
            pl.BlockSpec((h1d, h2d), lambda m: (0, 0)),
        ],
        out_specs=[
            pl.BlockSpec((ncj, _MB, 128), lambda m: (0, m, 0)),
            pl.BlockSpec((_MB, h2d), lambda m: (m, 0)),
        ],
        out_shape=[
            jax.ShapeDtypeStruct((ncj, n, 128), jnp.float32),
            jax.ShapeDtypeStruct((n, h2d), jnp.float32),
        ],
    )(x, aggp, cntp, w1l, b1, w1r, w2l, w2r)


def _tc_finalize(agg2p, cntp, q, b2):
    _, ncj, n, f = agg2p.shape
    h2d = ncj * 128
    grid = (n // _MB,)

    def body(aggp_ref, cntp_ref, q_ref, b2_ref, o_ref):
        cnt = cntp_ref[0, :, 0:1] + cntp_ref[1, :, 0:1]
        inv = 1.0 / jnp.maximum(cnt, 1.0)
        agg = jnp.concatenate(
            [aggp_ref[0, cj] + aggp_ref[1, cj] for cj in range(ncj)], axis=1)
        h = jnp.maximum(agg * inv + q_ref[...] + b2_ref[...], 0.0)
        nrm = jnp.sqrt(jnp.sum(h * h, axis=1, keepdims=True))
        o_ref[...] = h / jnp.maximum(nrm, 1e-12)

    return pl.pallas_call(
        body,
        grid=grid,
        in_specs=[
            pl.BlockSpec((_NC, ncj, _MB, 128), lambda m: (0, 0, m, 0)),
            pl.BlockSpec((_NC, _MB, 128), lambda m: (0, m, 0)),
            pl.BlockSpec((_MB, h2d), lambda m: (m, 0)),
            pl.BlockSpec((1, h2d), lambda m: (0, 0)),
        ],
        out_specs=pl.BlockSpec((_MB, h2d), lambda m: (m, 0)),
        out_shape=jax.ShapeDtypeStruct((n, h2d), jnp.float32),
    )(agg2p, cntp, q, b2)


def kernel(x, unused, edge_index, W1l, b1, W1r, W2l, b2, W2r):
    n, f = x.shape
    e = edge_index.shape[1]
    n_pad = ((n + 2 * _MB - 1) // (2 * _MB)) * (2 * _MB)
    x_p = jnp.pad(x, ((0, n_pad - n), (0, 0)))
    src_flat = edge_index[0]
    dst3d = edge_index[1].reshape(_NW, e // (_NW * _C), _C)
    rps = n_pad // _NS
    zeros_x = jnp.zeros((rps, f), jnp.float32)
    ones_c = jnp.ones((_C, 128), jnp.float32)

    aggp = _sc_agg_first(x_p, src_flat, dst3d, zeros_x)
    cntp = _sc_counts(dst3d, zeros_x, ones_c, n_pad)
    p8, q = _tc_layer1_project(x_p, aggp, cntp,
                               W1l.astype(jnp.bfloat16), b1.reshape(1, -1),
                               W1r.astype(jnp.bfloat16),
                               W2l.astype(jnp.bfloat16),
                               W2r.astype(jnp.bfloat16))
    agg2p = _sc_agg_second(p8, src_flat, dst3d, zeros_x)
    return _tc_finalize(agg2p, cntp, q, b2.reshape(1, -1))[:n]


# speedup vs baseline: 1.0008x; 1.0008x over previous
import functools

import jax
import jax.numpy as jnp
from jax import lax
from jax.experimental import pallas as pl
from jax.experimental.pallas import tpu as pltpu
from jax.experimental.pallas import tpu_sc as plsc

_NC = 2
_NS = 16
_NW = _NC * _NS
_C = 80
_MB = 512


def _sc_mesh():
    return plsc.VectorSubcoreMesh(core_axis_name="c", subcore_axis_name="s")


def _sub_slice(s, rps):
    return pl.ds(pl.multiple_of(s * rps, 8), rps)


_NB = 2


def _pipelined_edge_pass(table_hbm, src_v, dst_v, row_v, gsems, acc,
                         w_chunks, c_w):
    def src_sl(jj):
        return src_v.at[pl.ds(pl.multiple_of(jj * c_w, 8), c_w)]

    def start(jj, sub):
        pltpu.async_copy(table_hbm.at[src_sl(jj)], row_v.at[sub],
                         gsems[sub])

    def wait(jj, sub):
        pltpu.make_async_copy(table_hbm.at[src_sl(jj)], row_v.at[sub],
                              gsems[sub]).wait()

    for sub in range(_NB):
        start(sub, sub)

    main = (w_chunks // _NB) * _NB

    @pl.loop(0, main, step=_NB)
    def _(j):
        for sub in range(_NB):
            jj = j + sub
            wait(jj, sub)
            pltpu.sync_copy(row_v.at[sub], acc.at[dst_v.at[jj]], add=True)

            @pl.when(jj + _NB < w_chunks)
            def _():
                start(jj + _NB, sub)

    for sub in range(w_chunks - main):
        jj = main + sub
        wait(jj, sub)
        pltpu.sync_copy(row_v.at[sub], acc.at[dst_v.at[jj]], add=True)


def _sc_agg_first(x, src_flat, dst3d, zeros_x, ones_c):
    n, f = x.shape
    _, w_chunks, c_w = dst3d.shape
    ew = w_chunks * c_w
    rps = n // _NS

    @functools.partial(
        pl.kernel,
        out_type=[jax.ShapeDtypeStruct((_NC, n, f), jnp.float32),
                  jax.ShapeDtypeStruct((_NC, n, f), jnp.float32)],
        mesh=_sc_mesh(),
        scratch_types=[
            pltpu.VMEM_SHARED((n, f), jnp.float32),
            pltpu.VMEM((ew,), jnp.int32),
            pltpu.VMEM((w_chunks, c_w), jnp.int32),
            pltpu.VMEM((_NB, c_w, f), jnp.float32),
        ] + [pltpu.SemaphoreType.DMA] * _NB,
    )
    def k(x_hbm, src_hbm, dst_hbm, zx_hbm, ones_hbm,
          aggp_hbm, cntp_hbm, acc_x, src_v, dst_v, row_v, *gsems):
        c = lax.axis_index("c")
        s = lax.axis_index("s")
        wid = c * _NS + s
        sl = _sub_slice(s, rps)
        pltpu.sync_copy(zx_hbm, acc_x.at[sl])
        pltpu.sync_copy(
            src_hbm.at[pl.ds(pl.multiple_of(wid * ew, 8), ew)], src_v)
        pltpu.sync_copy(dst_hbm.at[wid], dst_v)
        plsc.subcore_barrier()

        _pipelined_edge_pass(x_hbm, src_v, dst_v, row_v, gsems, acc_x,
                             w_chunks, c_w)

        plsc.subcore_barrier()
        pltpu.sync_copy(acc_x.at[sl], aggp_hbm.at[c].at[sl])
        plsc.subcore_barrier()

        pltpu.sync_copy(zx_hbm, acc_x.at[sl])
        pltpu.sync_copy(ones_hbm, row_v.at[0])
        plsc.subcore_barrier()

        @pl.loop(0, w_chunks)
        def _(j):
            pltpu.sync_copy(row_v.at[0], acc_x.at[dst_v.at[j]], add=True)

        plsc.subcore_barrier()
        pltpu.sync_copy(acc_x.at[sl], cntp_hbm.at[c].at[sl])

    return k(x, src_flat, dst3d, zeros_x, ones_c)


def _sc_counts(dst3d, zeros_c, ones_c, n):
    _, w_chunks, c_w = dst3d.shape
    w_cnt = ones_c.shape[1]
    rps = n // _NS

    @functools.partial(
        pl.kernel,
        out_type=jax.ShapeDtypeStruct((_NC, n, w_cnt), jnp.float32),
        mesh=_sc_mesh(),
        scratch_types=[
            pltpu.VMEM_SHARED((n, w_cnt), jnp.float32),
            pltpu.VMEM((w_chunks, c_w), jnp.int32),
            pltpu.VMEM((c_w, w_cnt), jnp.float32),
        ],
    )
    def k(dst_hbm, zc_hbm, ones_hbm, cntp_hbm, acc_c, dst_v, ones_v):
        c = lax.axis_index("c")
        s = lax.axis_index("s")
        wid = c * _NS + s
        sl = _sub_slice(s, rps)
        pltpu.sync_copy(zc_hbm, acc_c.at[sl])
        pltpu.sync_copy(ones_hbm, ones_v)
        pltpu.sync_copy(dst_hbm.at[wid], dst_v)
        plsc.subcore_barrier()

        @pl.loop(0, w_chunks)
        def _(j):
            pltpu.sync_copy(ones_v, acc_c.at[dst_v.at[j]], add=True)

        plsc.subcore_barrier()
        pltpu.sync_copy(acc_c.at[sl], cntp_hbm.at[c].at[sl])

    return k(dst3d, zeros_c, ones_c)


def _sc_agg_second(p8, src_flat, dst3d, zeros_x):
    ncj, n, f = p8.shape
    _, w_chunks, c_w = dst3d.shape
    ew = w_chunks * c_w
    rps = n // _NS

    @functools.partial(
        pl.kernel,
        out_type=jax.ShapeDtypeStruct((_NC, ncj, n, f), jnp.float32),
        mesh=_sc_mesh(),
        scratch_types=[
            pltpu.VMEM_SHARED((n, f), jnp.float32),
            pltpu.VMEM((ew,), jnp.int32),
            pltpu.VMEM((w_chunks, c_w), jnp.int32),
            pltpu.VMEM((_NB, c_w, f), jnp.float32),
        ] + [pltpu.SemaphoreType.DMA] * _NB,
    )
    def k(p8_hbm, src_hbm, dst_hbm, zx_hbm, out_hbm,
          acc, src_v, dst_v, row_v, *gsems):
        c = lax.axis_index("c")
        s = lax.axis_index("s")
        wid = c * _NS + s
        sl = _sub_slice(s, rps)
        pltpu.sync_copy(
            src_hbm.at[pl.ds(pl.multiple_of(wid * ew, 8), ew)], src_v)
        pltpu.sync_copy(dst_hbm.at[wid], dst_v)
        for cj in range(ncj):
            pltpu.sync_copy(zx_hbm, acc.at[sl])
            plsc.subcore_barrier()

            _pipelined_edge_pass(p8_hbm.at[cj], src_v, dst_v, row_v,
                                 gsems, acc, w_chunks, c_w)

            plsc.subcore_barrier()
            pltpu.sync_copy(acc.at[sl], out_hbm.at[c].at[cj].at[sl])
            plsc.subcore_barrier()

    return k(p8, src_flat, dst3d, zeros_x)


def _tc_layer1_project(x, aggp, cntp, w1l, b1, w1r, w2l, w2r):
    n, f = x.shape
    h1d = w1l.shape[1]
    h2d = w2l.shape[1]
    ncj = h2d // 128
    grid = (n // _MB,)

    def body(x_ref, aggp_ref, cntp_ref, w1l_ref, b1_ref, w1r_ref,
             w2l_ref, w2r_ref, p8_ref, q_ref):
        cnt = cntp_ref[0, :, 0:1] + cntp_ref[1, :, 0:1]
        inv = 1.0 / jnp.maximum(cnt, 1.0)
        mean1 = (aggp_ref[0] + aggp_ref[1]) * inv
        h1 = jnp.maximum(
            jnp.dot(mean1, w1l_ref[...], preferred_element_type=jnp.float32)
            + jnp.dot(x_ref[...], w1r_ref[...], preferred_element_type=jnp.float32)
            + b1_ref[...], 0.0)
        q_ref[...] = jnp.dot(h1, w2r_ref[...], preferred_element_type=jnp.float32)
        p = jnp.dot(h1, w2l_ref[...], preferred_element_type=jnp.float32)
        for cj in range(ncj):
            p8_ref[cj] = p[:, cj * 128:(cj + 1) * 128]

    return pl.pallas_call(
        body,
        grid=grid,
        in_specs=[
            pl.BlockSpec((_MB, f), lambda m: (m, 0)),
            pl.BlockSpec((_NC, _MB, f), lambda m: (0, m, 0)),
            pl.BlockSpec((_NC, _MB, 128), lambda m: (0, m, 0)),
            pl.BlockSpec((f, h1d), lambda m: (0, 0)),
            pl.BlockSpec((1, h1d), lambda m: (0, 0)),
            pl.BlockSpec((f, h1d), lambda m: (0, 0)),
            pl.BlockSpec((h1d, h2d), lambda m: (0, 0)),
            pl.BlockSpec((h1d, h2d), lambda m: (0, 0)),
        ],
        out_specs=[
            pl.BlockSpec((ncj, _MB, 128), lambda m: (0, m, 0)),
            pl.BlockSpec((_MB, h2d), lambda m: (m, 0)),
        ],
        out_shape=[
            jax.ShapeDtypeStruct((ncj, n, 128), jnp.float32),
            jax.ShapeDtypeStruct((n, h2d), jnp.float32),
        ],
    )(x, aggp, cntp, w1l, b1, w1r, w2l, w2r)


def _tc_finalize(agg2p, cntp, q, b2):
    _, ncj, n, f = agg2p.shape
    h2d = ncj * 128
    grid = (n // _MB,)

    def body(aggp_ref, cntp_ref, q_ref, b2_ref, o_ref):
        cnt = cntp_ref[0, :, 0:1] + cntp_ref[1, :, 0:1]
        inv = 1.0 / jnp.maximum(cnt, 1.0)
        agg = jnp.concatenate(
            [aggp_ref[0, cj] + aggp_ref[1, cj] for cj in range(ncj)], axis=1)
        h = jnp.maximum(agg * inv + q_ref[...] + b2_ref[...], 0.0)
        nrm = jnp.sqrt(jnp.sum(h * h, axis=1, keepdims=True))
        o_ref[...] = h / jnp.maximum(nrm, 1e-12)

    return pl.pallas_call(
        body,
        grid=grid,
        in_specs=[
            pl.BlockSpec((_NC, ncj, _MB, 128), lambda m: (0, 0, m, 0)),
            pl.BlockSpec((_NC, _MB, 128), lambda m: (0, m, 0)),
            pl.BlockSpec((_MB, h2d), lambda m: (m, 0)),
            pl.BlockSpec((1, h2d), lambda m: (0, 0)),
        ],
        out_specs=pl.BlockSpec((_MB, h2d), lambda m: (m, 0)),
        out_shape=jax.ShapeDtypeStruct((n, h2d), jnp.float32),
    )(agg2p, cntp, q, b2)


def kernel(x, unused, edge_index, W1l, b1, W1r, W2l, b2, W2r):
    n, f = x.shape
    e = edge_index.shape[1]
    n_pad = ((n + 2 * _MB - 1) // (2 * _MB)) * (2 * _MB)
    x_p = jnp.pad(x, ((0, n_pad - n), (0, 0)))
    src_flat = edge_index[0]
    dst3d = edge_index[1].reshape(_NW, e // (_NW * _C), _C)
    rps = n_pad // _NS
    zeros_x = jnp.zeros((rps, f), jnp.float32)
    ones_c = jnp.ones((_C, 128), jnp.float32)

    aggp, cntp = _sc_agg_first(x_p, src_flat, dst3d, zeros_x, ones_c)
    p8, q = _tc_layer1_project(x_p, aggp, cntp, W1l, b1.reshape(1, -1), W1r,
                               W2l, W2r)
    agg2p = _sc_agg_second(p8, src_flat, dst3d, zeros_x)
    return _tc_finalize(agg2p, cntp, q, b2.reshape(1, -1))[:n]


# speedup vs baseline: 1.0178x; 1.0170x over previous
import functools

import jax
import jax.numpy as jnp
from jax import lax
from jax.experimental import pallas as pl
from jax.experimental.pallas import tpu as pltpu
from jax.experimental.pallas import tpu_sc as plsc

_NC = 2
_NS = 16
_NW = _NC * _NS
_C = 80
_MB = 512


def _sc_mesh():
    return plsc.VectorSubcoreMesh(core_axis_name="c", subcore_axis_name="s")


def _sub_slice(s, rps):
    return pl.ds(pl.multiple_of(s * rps, 8), rps)


_NB = 2


def _pipelined_edge_pass(table_hbm, src_v, dst_v, row_v, gsems, acc,
                         w_chunks, c_w):
    def src_sl(jj):
        return src_v.at[pl.ds(pl.multiple_of(jj * c_w, 8), c_w)]

    def start(jj, sub):
        pltpu.async_copy(table_hbm.at[src_sl(jj)], row_v.at[sub],
                         gsems[sub])

    def wait(jj, sub):
        pltpu.make_async_copy(table_hbm.at[src_sl(jj)], row_v.at[sub],
                              gsems[sub]).wait()

    for sub in range(_NB):
        start(sub, sub)

    main = (w_chunks // _NB) * _NB

    @pl.loop(0, main, step=_NB)
    def _(j):
        for sub in range(_NB):
            jj = j + sub
            wait(jj, sub)
            pltpu.sync_copy(row_v.at[sub], acc.at[dst_v.at[jj]], add=True)

            @pl.when(jj + _NB < w_chunks)
            def _():
                start(jj + _NB, sub)

    for sub in range(w_chunks - main):
        jj = main + sub
        wait(jj, sub)
        pltpu.sync_copy(row_v.at[sub], acc.at[dst_v.at[jj]], add=True)


def _sc_agg_first(x, src_flat, dst3d, zeros_x, ones_c):
    n, f = x.shape
    _, w_chunks, c_w = dst3d.shape
    ew = w_chunks * c_w
    rps = n // _NS

    @functools.partial(
        pl.kernel,
        out_type=[jax.ShapeDtypeStruct((_NC, n, f), jnp.float32),
                  jax.ShapeDtypeStruct((_NC, n, f), jnp.float32)],
        mesh=_sc_mesh(),
        scratch_types=[
            pltpu.VMEM_SHARED((n, f), jnp.float32),
            pltpu.VMEM((ew,), jnp.int32),
            pltpu.VMEM((w_chunks, c_w), jnp.int32),
            pltpu.VMEM((_NB, c_w, f), jnp.float32),
        ] + [pltpu.SemaphoreType.DMA] * _NB,
    )
    def k(x_hbm, src_hbm, dst_hbm, zx_hbm, ones_hbm,
          aggp_hbm, cntp_hbm, acc_x, src_v, dst_v, row_v, *gsems):
        c = lax.axis_index("c")
        s = lax.axis_index("s")
        wid = c * _NS + s
        sl = _sub_slice(s, rps)
        pltpu.sync_copy(zx_hbm, acc_x.at[sl])
        pltpu.sync_copy(
            src_hbm.at[pl.ds(pl.multiple_of(wid * ew, 8), ew)], src_v)
        pltpu.sync_copy(dst_hbm.at[wid], dst_v)
        plsc.subcore_barrier()

        _pipelined_edge_pass(x_hbm, src_v, dst_v, row_v, gsems, acc_x,
                             w_chunks, c_w)

        plsc.subcore_barrier()
        pltpu.sync_copy(acc_x.at[sl], aggp_hbm.at[c].at[sl])
        plsc.subcore_barrier()

        pltpu.sync_copy(zx_hbm, acc_x.at[sl])
        pltpu.sync_copy(ones_hbm, row_v.at[0])
        plsc.subcore_barrier()

        @pl.loop(0, w_chunks)
        def _(j):
            pltpu.sync_copy(row_v.at[0], acc_x.at[dst_v.at[j]], add=True)

        plsc.subcore_barrier()
        pltpu.sync_copy(acc_x.at[sl], cntp_hbm.at[c].at[sl])

    return k(x, src_flat, dst3d, zeros_x, ones_c)


def _sc_counts(dst3d, zeros_c, ones_c, n):
    _, w_chunks, c_w = dst3d.shape
    w_cnt = ones_c.shape[1]
    rps = n // _NS

    @functools.partial(
        pl.kernel,
        out_type=jax.ShapeDtypeStruct((_NC, n, w_cnt), jnp.float32),
        mesh=_sc_mesh(),
        scratch_types=[
            pltpu.VMEM_SHARED((n, w_cnt), jnp.float32),
            pltpu.VMEM((w_chunks, c_w), jnp.int32),
            pltpu.VMEM((c_w, w_cnt), jnp.float32),
        ],
    )
    def k(dst_hbm, zc_hbm, ones_hbm, cntp_hbm, acc_c, dst_v, ones_v):
        c = lax.axis_index("c")
        s = lax.axis_index("s")
        wid = c * _NS + s
        sl = _sub_slice(s, rps)
        pltpu.sync_copy(zc_hbm, acc_c.at[sl])
        pltpu.sync_copy(ones_hbm, ones_v)
        pltpu.sync_copy(dst_hbm.at[wid], dst_v)
        plsc.subcore_barrier()

        @pl.loop(0, w_chunks)
        def _(j):
            pltpu.sync_copy(ones_v, acc_c.at[dst_v.at[j]], add=True)

        plsc.subcore_barrier()
        pltpu.sync_copy(acc_c.at[sl], cntp_hbm.at[c].at[sl])

    return k(dst3d, zeros_c, ones_c)


def _sc_agg_second(p8, src_flat, dst3d, zeros_x):
    ncj, n, f = p8.shape
    _, w_chunks, c_w = dst3d.shape
    ew = w_chunks * c_w
    rps = n // _NS

    @functools.partial(
        pl.kernel,
        out_type=jax.ShapeDtypeStruct((_NC, ncj, n, f), jnp.float32),
        mesh=_sc_mesh(),
        scratch_types=[
            pltpu.VMEM_SHARED((n, f), jnp.float32),
            pltpu.VMEM((ew,), jnp.int32),
            pltpu.VMEM((w_chunks, c_w), jnp.int32),
            pltpu.VMEM((_NB, c_w, f), jnp.float32),
        ] + [pltpu.SemaphoreType.DMA] * _NB,
    )
    def k(p8_hbm, src_hbm, dst_hbm, zx_hbm, out_hbm,
          acc, src_v, dst_v, row_v, *gsems):
        c = lax.axis_index("c")
        s = lax.axis_index("s")
        wid = c * _NS + s
        sl = _sub_slice(s, rps)
        pltpu.sync_copy(
            src_hbm.at[pl.ds(pl.multiple_of(wid * ew, 8), ew)], src_v)
        pltpu.sync_copy(dst_hbm.at[wid], dst_v)
        for cj in range(ncj):
            pltpu.sync_copy(zx_hbm, acc.at[sl])
            plsc.subcore_barrier()

            _pipelined_edge_pass(p8_hbm.at[cj], src_v, dst_v, row_v,
                                 gsems, acc, w_chunks, c_w)

            plsc.subcore_barrier()
            pltpu.sync_copy(acc.at[sl], out_hbm.at[c].at[cj].at[sl])
            plsc.subcore_barrier()

    return k(p8, src_flat, dst3d, zeros_x)


def _tc_layer1_project(x, aggp, cntp, w1l, b1, w1r, w2l):
    n, f = x.shape
    h1d = w1l.shape[1]
    h2d = w2l.shape[1]
    ncj = h2d // 128
    grid = (n // _MB,)

    def body(x_ref, aggp_ref, cntp_ref, w1l_ref, b1_ref, w1r_ref,
             w2l_ref, p8_ref, h1_ref):
        cnt = cntp_ref[0, :, 0:1] + cntp_ref[1, :, 0:1]
        inv = 1.0 / jnp.maximum(cnt, 1.0)
        mean1 = (aggp_ref[0] + aggp_ref[1]) * inv
        h1 = jnp.maximum(
            jnp.dot(mean1, w1l_ref[...], preferred_element_type=jnp.float32)
            + jnp.dot(x_ref[...], w1r_ref[...], preferred_element_type=jnp.float32)
            + b1_ref[...], 0.0)
        h1_ref[...] = h1
        p = jnp.dot(h1, w2l_ref[...], preferred_element_type=jnp.float32)
        for cj in range(ncj):
            p8_ref[cj] = p[:, cj * 128:(cj + 1) * 128]

    return pl.pallas_call(
        body,
        grid=grid,
        in_specs=[
            pl.BlockSpec((_MB, f), lambda m: (m, 0)),
            pl.BlockSpec((_NC, _MB, f), lambda m: (0, m, 0)),
            pl.BlockSpec((_NC, _MB, 128), lambda m: (0, m, 0)),
            pl.BlockSpec((f, h1d), lambda m: (0, 0)),
            pl.BlockSpec((1, h1d), lambda m: (0, 0)),
            pl.BlockSpec((f, h1d), lambda m: (0, 0)),
            pl.BlockSpec((h1d, h2d), lambda m: (0, 0)),
        ],
        out_specs=[
            pl.BlockSpec((ncj, _MB, 128), lambda m: (0, m, 0)),
            pl.BlockSpec((_MB, h1d), lambda m: (m, 0)),
        ],
        out_shape=[
            jax.ShapeDtypeStruct((ncj, n, 128), jnp.float32),
            jax.ShapeDtypeStruct((n, h1d), jnp.float32),
        ],
    )(x, aggp, cntp, w1l, b1, w1r, w2l)


def _tc_q(h1, w2r):
    n, h1d = h1.shape
    h2d = w2r.shape[1]
    grid = (n // _MB,)

    def body(h1_ref, w2r_ref, q_ref):
        q_ref[...] = jnp.dot(h1_ref[...], w2r_ref[...],
                             preferred_element_type=jnp.float32)

    return pl.pallas_call(
        body,
        grid=grid,
        in_specs=[
            pl.BlockSpec((_MB, h1d), lambda m: (m, 0)),
            pl.BlockSpec((h1d, h2d), lambda m: (0, 0)),
        ],
        out_specs=pl.BlockSpec((_MB, h2d), lambda m: (m, 0)),
        out_shape=jax.ShapeDtypeStruct((n, h2d), jnp.float32),
    )(h1, w2r)


def _tc_finalize(agg2p, cntp, q, b2):
    _, ncj, n, f = agg2p.shape
    h2d = ncj * 128
    grid = (n // _MB,)

    def body(aggp_ref, cntp_ref, q_ref, b2_ref, o_ref):
        cnt = cntp_ref[0, :, 0:1] + cntp_ref[1, :, 0:1]
        inv = 1.0 / jnp.maximum(cnt, 1.0)
        agg = jnp.concatenate(
            [aggp_ref[0, cj] + aggp_ref[1, cj] for cj in range(ncj)], axis=1)
        h = jnp.maximum(agg * inv + q_ref[...] + b2_ref[...], 0.0)
        nrm = jnp.sqrt(jnp.sum(h * h, axis=1, keepdims=True))
        o_ref[...] = h / jnp.maximum(nrm, 1e-12)

    return pl.pallas_call(
        body,
        grid=grid,
        in_specs=[
            pl.BlockSpec((_NC, ncj, _MB, 128), lambda m: (0, 0, m, 0)),
            pl.BlockSpec((_NC, _MB, 128), lambda m: (0, m, 0)),
            pl.BlockSpec((_MB, h2d), lambda m: (m, 0)),
            pl.BlockSpec((1, h2d), lambda m: (0, 0)),
        ],
        out_specs=pl.BlockSpec((_MB, h2d), lambda m: (m, 0)),
        out_shape=jax.ShapeDtypeStruct((n, h2d), jnp.float32),
    )(agg2p, cntp, q, b2)


def kernel(x, unused, edge_index, W1l, b1, W1r, W2l, b2, W2r):
    n, f = x.shape
    e = edge_index.shape[1]
    n_pad = ((n + 2 * _MB - 1) // (2 * _MB)) * (2 * _MB)
    x_p = jnp.pad(x, ((0, n_pad - n), (0, 0)))
    src_flat = edge_index[0]
    dst3d = edge_index[1].reshape(_NW, e // (_NW * _C), _C)
    rps = n_pad // _NS
    zeros_x = jnp.zeros((rps, f), jnp.float32)
    ones_c = jnp.ones((_C, 128), jnp.float32)

    aggp, cntp = _sc_agg_first(x_p, src_flat, dst3d, zeros_x, ones_c)
    p8, h1 = _tc_layer1_project(x_p, aggp, cntp, W1l, b1.reshape(1, -1),
                                W1r, W2l)
    q = _tc_q(h1, W2r)
    agg2p = _sc_agg_second(p8, src_flat, dst3d, zeros_x)
    return _tc_finalize(agg2p, cntp, q, b2.reshape(1, -1))[:n]


# speedup vs baseline: 1.0180x; 1.0002x over previous
import functools

import jax
import jax.numpy as jnp
from jax import lax
from jax.experimental import pallas as pl
from jax.experimental.pallas import tpu as pltpu
from jax.experimental.pallas import tpu_sc as plsc

_NC = 2
_NS = 16
_NW = _NC * _NS
_C = 80
_MB = 512


def _sc_mesh():
    return plsc.VectorSubcoreMesh(core_axis_name="c", subcore_axis_name="s")


def _sub_slice(s, rps):
    return pl.ds(pl.multiple_of(s * rps, 8), rps)


_NB = 2


def _pipelined_edge_pass(table_hbm, src_v, dst_v, row_v, gsems, acc,
                         w_chunks, c_w):
    def src_sl(jj):
        return src_v.at[pl.ds(pl.multiple_of(jj * c_w, 8), c_w)]

    def start(jj, sub):
        pltpu.async_copy(table_hbm.at[src_sl(jj)], row_v.at[sub],
                         gsems[sub])

    def wait(jj, sub):
        pltpu.make_async_copy(table_hbm.at[src_sl(jj)], row_v.at[sub],
                              gsems[sub]).wait()

    for sub in range(_NB):
        start(sub, sub)

    main = (w_chunks // _NB) * _NB

    @pl.loop(0, main, step=_NB)
    def _(j):
        for sub in range(_NB):
            jj = j + sub
            wait(jj, sub)
            pltpu.sync_copy(row_v.at[sub], acc.at[dst_v.at[jj]], add=True)

            @pl.when(jj + _NB < w_chunks)
            def _():
                start(jj + _NB, sub)

    for sub in range(w_chunks - main):
        jj = main + sub
        wait(jj, sub)
        pltpu.sync_copy(row_v.at[sub], acc.at[dst_v.at[jj]], add=True)


def _sc_agg_first(x, src_flat, dst3d, zeros_x, ones_c):
    n, f = x.shape
    _, w_chunks, c_w = dst3d.shape
    ew = w_chunks * c_w
    rps = n // _NS

    @functools.partial(
        pl.kernel,
        out_type=[jax.ShapeDtypeStruct((_NC, n, f), jnp.float32),
                  jax.ShapeDtypeStruct((_NC, n, f), jnp.float32)],
        mesh=_sc_mesh(),
        scratch_types=[
            pltpu.VMEM_SHARED((n, f), jnp.float32),
            pltpu.VMEM((ew,), jnp.int32),
            pltpu.VMEM((w_chunks, c_w), jnp.int32),
            pltpu.VMEM((_NB, c_w, f), jnp.float32),
        ] + [pltpu.SemaphoreType.DMA] * _NB,
    )
    def k(x_hbm, src_hbm, dst_hbm, zx_hbm, ones_hbm,
          aggp_hbm, cntp_hbm, acc_x, src_v, dst_v, row_v, *gsems):
        c = lax.axis_index("c")
        s = lax.axis_index("s")
        wid = c * _NS + s
        sl = _sub_slice(s, rps)
        pltpu.sync_copy(zx_hbm, acc_x.at[sl])
        pltpu.sync_copy(
            src_hbm.at[pl.ds(pl.multiple_of(wid * ew, 8), ew)], src_v)
        pltpu.sync_copy(dst_hbm.at[wid], dst_v)
        plsc.subcore_barrier()

        _pipelined_edge_pass(x_hbm, src_v, dst_v, row_v, gsems, acc_x,
                             w_chunks, c_w)

        plsc.subcore_barrier()
        pltpu.sync_copy(acc_x.at[sl], aggp_hbm.at[c].at[sl])
        plsc.subcore_barrier()

        pltpu.sync_copy(zx_hbm, acc_x.at[sl])
        pltpu.sync_copy(ones_hbm, row_v.at[0])
        plsc.subcore_barrier()

        @pl.loop(0, w_chunks)
        def _(j):
            pltpu.sync_copy(row_v.at[0], acc_x.at[dst_v.at[j]], add=True)

        plsc.subcore_barrier()
        pltpu.sync_copy(acc_x.at[sl], cntp_hbm.at[c].at[sl])

    return k(x, src_flat, dst3d, zeros_x, ones_c)


def _sc_agg_second(p8, src_flat, dst3d, zeros_x):
    ncj, n, f = p8.shape
    _, w_chunks, c_w = dst3d.shape
    ew = w_chunks * c_w
    rps = n // _NS

    @functools.partial(
        pl.kernel,
        out_type=jax.ShapeDtypeStruct((_NC, ncj, n, f), jnp.float32),
        mesh=_sc_mesh(),
        scratch_types=[
            pltpu.VMEM_SHARED((n, f), jnp.float32),
            pltpu.VMEM((ew,), jnp.int32),
            pltpu.VMEM((w_chunks, c_w), jnp.int32),
            pltpu.VMEM((_NB, c_w, f), jnp.float32),
        ] + [pltpu.SemaphoreType.DMA] * _NB,
    )
    def k(p8_hbm, src_hbm, dst_hbm, zx_hbm, out_hbm,
          acc, src_v, dst_v, row_v, *gsems):
        c = lax.axis_index("c")
        s = lax.axis_index("s")
        wid = c * _NS + s
        sl = _sub_slice(s, rps)
        pltpu.sync_copy(
            src_hbm.at[pl.ds(pl.multiple_of(wid * ew, 8), ew)], src_v)
        pltpu.sync_copy(dst_hbm.at[wid], dst_v)
        for cj in range(ncj):
            pltpu.sync_copy(zx_hbm, acc.at[sl])
            plsc.subcore_barrier()

            _pipelined_edge_pass(p8_hbm.at[cj], src_v, dst_v, row_v,
                                 gsems, acc, w_chunks, c_w)

            plsc.subcore_barrier()
            pltpu.sync_copy(acc.at[sl], out_hbm.at[c].at[cj].at[sl])
            plsc.subcore_barrier()

    return k(p8, src_flat, dst3d, zeros_x)


def _tc_layer1_project(x, aggp, cntp, w1l, b1, w1r, w2l):
    n, f = x.shape
    h1d = w1l.shape[1]
    h2d = w2l.shape[1]
    ncj = h2d // 128
    grid = (n // _MB,)

    def body(x_ref, aggp_ref, cntp_ref, w1l_ref, b1_ref, w1r_ref,
             w2l_ref, p8_ref, h1_ref):
        cnt = cntp_ref[0, :, 0:1] + cntp_ref[1, :, 0:1]
        inv = 1.0 / jnp.maximum(cnt, 1.0)
        mean1 = (aggp_ref[0] + aggp_ref[1]) * inv
        h1 = jnp.maximum(
            jnp.dot(mean1, w1l_ref[...], preferred_element_type=jnp.float32)
            + jnp.dot(x_ref[...], w1r_ref[...], preferred_element_type=jnp.float32)
            + b1_ref[...], 0.0)
        h1_ref[...] = h1
        p = jnp.dot(h1, w2l_ref[...], preferred_element_type=jnp.float32)
        for cj in range(ncj):
            p8_ref[cj] = p[:, cj * 128:(cj + 1) * 128]

    return pl.pallas_call(
        body,
        grid=grid,
        in_specs=[
            pl.BlockSpec((_MB, f), lambda m: (m, 0)),
            pl.BlockSpec((_NC, _MB, f), lambda m: (0, m, 0)),
            pl.BlockSpec((_NC, _MB, 128), lambda m: (0, m, 0)),
            pl.BlockSpec((f, h1d), lambda m: (0, 0)),
            pl.BlockSpec((1, h1d), lambda m: (0, 0)),
            pl.BlockSpec((f, h1d), lambda m: (0, 0)),
            pl.BlockSpec((h1d, h2d), lambda m: (0, 0)),
        ],
        out_specs=[
            pl.BlockSpec((ncj, _MB, 128), lambda m: (0, m, 0)),
            pl.BlockSpec((_MB, h1d), lambda m: (m, 0)),
        ],
        out_shape=[
            jax.ShapeDtypeStruct((ncj, n, 128), jnp.float32),
            jax.ShapeDtypeStruct((n, h1d), jnp.float32),
        ],
    )(x, aggp, cntp, w1l, b1, w1r, w2l)


def _tc_q(h1, w2r):
    n, h1d = h1.shape
    h2d = w2r.shape[1]
    grid = (n // _MB,)

    def body(h1_ref, w2r_ref, q_ref):
        q_ref[...] = jnp.dot(h1_ref[...], w2r_ref[...],
                             preferred_element_type=jnp.float32)

    return pl.pallas_call(
        body,
        grid=grid,
        in_specs=[
            pl.BlockSpec((_MB, h1d), lambda m: (m, 0)),
            pl.BlockSpec((h1d, h2d), lambda m: (0, 0)),
        ],
        out_specs=pl.BlockSpec((_MB, h2d), lambda m: (m, 0)),
        out_shape=jax.ShapeDtypeStruct((n, h2d), jnp.float32),
    )(h1, w2r)


def _tc_finalize(agg2p, cntp, q, b2):
    _, ncj, n, f = agg2p.shape
    h2d = ncj * 128
    grid = (n // _MB,)

    def body(aggp_ref, cntp_ref, q_ref, b2_ref, o_ref):
        cnt = cntp_ref[0, :, 0:1] + cntp_ref[1, :, 0:1]
        inv = 1.0 / jnp.maximum(cnt, 1.0)
        agg = jnp.concatenate(
            [aggp_ref[0, cj] + aggp_ref[1, cj] for cj in range(ncj)], axis=1)
        h = jnp.maximum(agg * inv + q_ref[...] + b2_ref[...], 0.0)
        nrm = jnp.sqrt(jnp.sum(h * h, axis=1, keepdims=True))
        o_ref[...] = h / jnp.maximum(nrm, 1e-12)

    return pl.pallas_call(
        body,
        grid=grid,
        in_specs=[
            pl.BlockSpec((_NC, ncj, _MB, 128), lambda m: (0, 0, m, 0)),
            pl.BlockSpec((_NC, _MB, 128), lambda m: (0, m, 0)),
            pl.BlockSpec((_MB, h2d), lambda m: (m, 0)),
            pl.BlockSpec((1, h2d), lambda m: (0, 0)),
        ],
        out_specs=pl.BlockSpec((_MB, h2d), lambda m: (m, 0)),
        out_shape=jax.ShapeDtypeStruct((n, h2d), jnp.float32),
    )(agg2p, cntp, q, b2)


def kernel(x, unused, edge_index, W1l, b1, W1r, W2l, b2, W2r):
    n, f = x.shape
    e = edge_index.shape[1]
    n_pad = ((n + 2 * _MB - 1) // (2 * _MB)) * (2 * _MB)
    x_p = jnp.pad(x, ((0, n_pad - n), (0, 0)))
    src_flat = edge_index[0]
    dst3d = edge_index[1].reshape(_NW, e // (_NW * _C), _C)
    rps = n_pad // _NS
    zeros_x = jnp.zeros((rps, f), jnp.float32)
    ones_c = jnp.ones((_C, 128), jnp.float32)

    aggp, cntp = _sc_agg_first(x_p, src_flat, dst3d, zeros_x, ones_c)
    p8, h1 = _tc_layer1_project(x_p, aggp, cntp, W1l, b1.reshape(1, -1),
                                W1r, W2l)
    q = _tc_q(h1, W2r)
    agg2p = _sc_agg_second(p8, src_flat, dst3d, zeros_x)
    return _tc_finalize(agg2p, cntp, q, b2.reshape(1, -1))[:n]
